# baseline (speedup 1.0000x reference)
import functools

import jax
import jax.numpy as jnp
from jax import lax
from jax.experimental import pallas as pl
from jax.experimental.pallas import tpu as pltpu
from jax.experimental.pallas import tpu_sc as plsc

N = 50000
E = 800000
F_IN = 840
H = 64
HH = 32
G = 128
EPR = 128
NROWS = E // EPR
NS = 16
NC = 2

_mesh = plsc.VectorSubcoreMesh(core_axis_name="c", subcore_axis_name="s")
_sc_params = pltpu.CompilerParams(use_tc_tiling_on_sc=False)


def _fill(ref, n, val):
  v = jnp.full((16,), val, ref.dtype)
  def body(i, _):
    ref[pl.ds(i * 16, 16)] = v
    return 0
  lax.fori_loop(0, n // 16, body, 0)


def _fill2d(ref, rows, val):
  v = jnp.full((16,), val, ref.dtype)
  def body(i, _):
    ref[i, pl.ds(0, 16)] = v
    ref[i, pl.ds(16, 16)] = v
    return 0
  lax.fori_loop(0, rows, body, 0)


def _deg_body(ei3, deg0, deg1, dstb, ones_v, zb, deg_sh,
              si0, si1, si2, si3, ss0, ss1, ss2, ss3, ss4, ss5, ss6, ss7):
  c = lax.axis_index("c")
  s = lax.axis_index("s")
  w = s * NC + c
  si = (si0, si1, si2, si3)
  ss = (ss0, ss1, ss2, ss3, ss4, ss5, ss6, ss7)

  _fill(ones_v, EPR, 1.0)
  _fill(zb, EPR, 0.0)

  for k in range(25):
    @pl.when((s < 15) | (k < 15))
    def _():
      pltpu.sync_copy(zb, deg_sh.at[pl.ds(s * 3200 + k * 128, 128)])
  @pl.when(s == 15)
  def _():
    pltpu.sync_copy(zb.at[pl.ds(0, 80)], deg_sh.at[pl.ds(49920, 80)])
  plsc.subcore_barrier()

  nr = jnp.where(w < 10, 196, 195)
  start = jnp.where(w < 10, 196 * w, 195 * w + 10)
  row_of = lambda j: start + jnp.minimum(j, nr - 1)

  def issue_idx(j, slot, sem):
    pltpu.async_copy(ei3.at[1, pl.ds(row_of(j), 1), :],
                     dstb.at[pl.ds(slot, 1), :], sem)

  def wait_idx(slot, sem):
    pltpu.make_async_copy(ei3.at[1, pl.ds(0, 1), :],
                          dstb.at[pl.ds(slot, 1), :], sem).wait()

  def wait_sc(slot, sem):
    pltpu.make_async_copy(ones_v, deg_sh.at[dstb.at[slot]], sem).wait()

  for b in range(4):
    issue_idx(jnp.int32(b), b, si[b])

  NRS = 200

  def group8(g, _):
    for b in range(8):
      j = g * 8 + b
      wait_idx(b, si[b % 4])
      @pl.when(j < nr)
      def _():
        pltpu.async_copy(ones_v, deg_sh.at[dstb.at[b]], ss[b], add=True)
      @pl.when((j >= 4) & (j - 4 < nr))
      def _():
        wait_sc((b + 4) % 8, ss[(b + 4) % 8])
      issue_idx(j + 4, (b + 4) % 8, si[b % 4])
    return 0
  lax.fori_loop(0, NRS // 8, group8, 0)

  for b in range(4):
    wait_idx(b, si[b])

  plsc.subcore_barrier()

  def copy_out(dst_ref):
    def cbody(k, _):
      o = s * 3200 + k * 128
      pltpu.sync_copy(deg_sh.at[pl.ds(o, 128)], zb)
      pltpu.sync_copy(zb, dst_ref.at[pl.ds(o, 128)])
      return 0
    nchunk = jnp.where(s < 15, 25, 15)
    lax.fori_loop(0, nchunk, cbody, 0)
    @pl.when(s == 15)
    def _():
      pltpu.sync_copy(deg_sh.at[pl.ds(49920, 80)], zb.at[pl.ds(0, 80)])
      pltpu.sync_copy(zb.at[pl.ds(0, 80)], dst_ref.at[pl.ds(49920, 80)])

  @pl.when(c == 0)
  def _():
    copy_out(deg0)
  @pl.when(c == 1)
  def _():
    copy_out(deg1)


_deg_kernel = functools.partial(
    pl.kernel, _deg_body,
    out_type=(jax.ShapeDtypeStruct((N,), jnp.float32),
              jax.ShapeDtypeStruct((N,), jnp.float32)),
    mesh=_mesh,
    scratch_types=[
        pltpu.VMEM((8, EPR), jnp.int32),
        pltpu.VMEM((EPR,), jnp.float32),
        pltpu.VMEM((EPR,), jnp.float32),
        pltpu.VMEM_SHARED((N,), jnp.float32),
    ] + [pltpu.SemaphoreType.DMA] * 12,
    compiler_params=_sc_params,
)()


NRE = 392


def _agg_body(ei3, ylo, yhi, outlo, outhi, srcb, dstb, rows, zbuf, agg_sh,
              sg0, sg1, sg2, sg3, ss0, ss1, ss2, ss3, si0, si1, si2, si3):
  c = lax.axis_index("c")
  s = lax.axis_index("s")
  sg = (sg0, sg1, sg2, sg3)
  ss = (ss0, ss1, ss2, ss3)
  si = (si0, si1, si2, si3)

  _fill2d(zbuf, EPR, 0.0)

  r0 = s * 3128
  def zbody(k, _):
    pltpu.sync_copy(zbuf, agg_sh.at[pl.ds(r0 + k * 128, 128), :])
    return 0
  lax.fori_loop(0, 24, zbody, 0)
  @pl.when(s < 15)
  def _():
    pltpu.sync_copy(zbuf.at[pl.ds(0, 56), :],
                    agg_sh.at[pl.ds(r0 + 3072, 56), :])
  @pl.when(s == 15)
  def _():
    pltpu.sync_copy(zbuf.at[pl.ds(0, 8), :],
                    agg_sh.at[pl.ds(r0 + 3072, 8), :])
  plsc.subcore_barrier()

  nr = jnp.where(s < 10, 391, 390)
  start = jnp.where(s < 10, 391 * s, 390 * s + 10)
  row_of = lambda j: start + jnp.minimum(j, nr - 1)

  def do_edges(y_ref):
    def issue_idx(j, slot, sem):
      pltpu.async_copy(ei3.at[0, pl.ds(row_of(j), 1), :],
                       srcb.at[pl.ds(slot, 1), :], sem)
      pltpu.async_copy(ei3.at[1, pl.ds(row_of(j), 1), :],
                       dstb.at[pl.ds(slot, 1), :], sem)

    def wait_idx(slot, sem):
      pltpu.make_async_copy(ei3.at[0, pl.ds(0, 1), :],
                            srcb.at[pl.ds(slot, 1), :], sem).wait()
      pltpu.make_async_copy(ei3.at[1, pl.ds(0, 1), :],
                            dstb.at[pl.ds(slot, 1), :], sem).wait()

    def issue_gather(islot, rslot, sem):
      pltpu.async_copy(y_ref.at[srcb.at[islot]], rows.at[rslot], sem)

    def wait_gather(islot, rslot, sem):
      pltpu.make_async_copy(y_ref.at[srcb.at[islot]], rows.at[rslot],
                            sem).wait()

    def issue_sc(rslot, islot, sem):
      pltpu.async_copy(rows.at[rslot], agg_sh.at[dstb.at[islot]], sem,
                       add=True)

    def wait_sc(rslot, islot, sem):
      pltpu.make_async_copy(rows.at[rslot], agg_sh.at[dstb.at[islot]],
                            sem).wait()

    for b in range(3):
      issue_idx(jnp.int32(b), b, si[b])
    for b in range(3):
      wait_idx(b, si[b])
      issue_gather(b, b, sg[b])
    issue_idx(jnp.int32(3), 3, si[3])

    def group8(g, _):
      for b in range(8):
        j = g * 8 + b
        wait_gather(b, b % 4, sg[b % 4])
        @pl.when((j >= 1) & (j - 1 < nr))
        def _():
          wait_sc((b + 3) % 4, (b + 7) % 8, ss[(b + 3) % 4])
        wait_idx((b + 3) % 8, si[(b + 3) % 4])
        issue_gather((b + 3) % 8, (b + 3) % 4, sg[(b + 3) % 4])
        @pl.when(j < nr)
        def _():
          issue_sc(b % 4, b, ss[b % 4])
        issue_idx(j + 4, (b + 4) % 8, si[b % 4])
      return 0
    lax.fori_loop(0, NRE // 8, group8, 0)

    for b in range(3):
      wait_gather(b, b % 4, sg[b % 4])
    wait_idx(3, si[3])

  @pl.when(c == 0)
  def _():
    do_edges(ylo)
  @pl.when(c == 1)
  def _():
    do_edges(yhi)

  plsc.subcore_barrier()

  def copy_out(dst_ref):
    def cbody(k, _):
      o = r0 + k * 128
      pltpu.sync_copy(agg_sh.at[pl.ds(o, 128), :], zbuf)
      pltpu.sync_copy(zbuf, dst_ref.at[pl.ds(o, 128), :])
      return 0
    lax.fori_loop(0, 24, cbody, 0)
    @pl.when(s < 15)
    def _():
      pltpu.sync_copy(agg_sh.at[pl.ds(r0 + 3072, 56), :],
                      zbuf.at[pl.ds(0, 56), :])
      pltpu.sync_copy(zbuf.at[pl.ds(0, 56), :],
                      dst_ref.at[pl.ds(r0 + 3072, 56), :])
    @pl.when(s == 15)
    def _():
      pltpu.sync_copy(agg_sh.at[pl.ds(r0 + 3072, 8), :],
                      zbuf.at[pl.ds(0, 8), :])
      pltpu.sync_copy(zbuf.at[pl.ds(0, 8), :],
                      dst_ref.at[pl.ds(r0 + 3072, 8), :])

  @pl.when(c == 0)
  def _():
    copy_out(outlo)
  @pl.when(c == 1)
  def _():
    copy_out(outhi)


_agg_kernel = functools.partial(
    pl.kernel, _agg_body,
    out_type=(jax.ShapeDtypeStruct((N, HH), jnp.float32),
              jax.ShapeDtypeStruct((N, HH), jnp.float32)),
    mesh=_mesh,
    scratch_types=[
        pltpu.VMEM((8, EPR), jnp.int32),
        pltpu.VMEM((8, EPR), jnp.int32),
        pltpu.VMEM((4, EPR, HH), jnp.float32),
        pltpu.VMEM((EPR, HH), jnp.float32),
        pltpu.VMEM_SHARED((N, HH), jnp.float32),
    ] + [pltpu.SemaphoreType.DMA] * 12,
    compiler_params=_sc_params,
)()


_BN = 400
_NB = N // _BN


def _tocol(row):
  ii = lax.broadcasted_iota(jnp.int32, (_BN, _BN), 0)
  jj = lax.broadcasted_iota(jnp.int32, (_BN, _BN), 1)
  ident = jnp.where(ii == jj, 1.0, 0.0)
  return lax.dot_general(ident, row, (((1,), (1,)), ((), ())),
                         preferred_element_type=jnp.float32)


def _mm1_body(x_ref, w_ref, d0_ref, d1_ref, ylo_ref, yhi_ref, dinv_ref):
  deg = 1.0 + d0_ref[...][0] + d1_ref[...][0]
  dinv = lax.rsqrt(deg)
  dvc = _tocol(dinv)
  xw = jnp.dot(x_ref[...], w_ref[...], preferred_element_type=jnp.float32)
  y = xw * dvc
  ylo_ref[...] = y[:, :HH]
  yhi_ref[...] = y[:, HH:]
  dinv_ref[...] = dinv[None]


def _mm1(xv, w1, d0, d1):
  return pl.pallas_call(
      _mm1_body,
      grid=(_NB,),
      in_specs=[
          pl.BlockSpec((_BN, F_IN), lambda i: (i, 0)),
          pl.BlockSpec((F_IN, H), lambda i: (0, 0)),
          pl.BlockSpec((1, 1, _BN), lambda i: (i, 0, 0)),
          pl.BlockSpec((1, 1, _BN), lambda i: (i, 0, 0)),
      ],
      out_specs=[
          pl.BlockSpec((_BN, HH), lambda i: (i, 0)),
          pl.BlockSpec((_BN, HH), lambda i: (i, 0)),
          pl.BlockSpec((1, 1, _BN), lambda i: (i, 0, 0)),
      ],
      out_shape=[
          jax.ShapeDtypeStruct((N, HH), jnp.float32),
          jax.ShapeDtypeStruct((N, HH), jnp.float32),
          jax.ShapeDtypeStruct((_NB, 1, _BN), jnp.float32),
      ],
  )(xv, w1, d0, d1)


def _comb_body(alo, ahi, ylo, yhi, dinv, b_ref, w_ref, olo, ohi):
  dv = _tocol(dinv[...][0])
  t = jnp.concatenate([alo[...] + ylo[...], ahi[...] + yhi[...]], axis=1)
  h = jnp.maximum(dv * t + b_ref[...], 0.0)
  yn = jnp.dot(h, w_ref[...], preferred_element_type=jnp.float32) * dv
  olo[...] = yn[:, :HH]
  ohi[...] = yn[:, HH:]


def _combine(alo, ahi, ylo, yhi, dinv, b, w):
  spec = pl.BlockSpec((_BN, HH), lambda i: (i, 0))
  return pl.pallas_call(
      _comb_body,
      grid=(_NB,),
      in_specs=[
          spec, spec, spec, spec,
          pl.BlockSpec((1, 1, _BN), lambda i: (i, 0, 0)),
          pl.BlockSpec((1, H), lambda i: (0, 0)),
          pl.BlockSpec((H, H), lambda i: (0, 0)),
      ],
      out_specs=[spec, spec],
      out_shape=[
          jax.ShapeDtypeStruct((N, HH), jnp.float32),
          jax.ShapeDtypeStruct((N, HH), jnp.float32),
      ],
  )(alo, ahi, ylo, yhi, dinv, b, w)


def _final_body(alo, ahi, ylo, yhi, dinv, b_ref, batch, wl, bl, out_ref, acc):
  i = pl.program_id(0)

  @pl.when(i == 0)
  def _():
    acc[...] = jnp.zeros((G, H + 1), jnp.float32)

  dv = _tocol(dinv[...][0])
  t = jnp.concatenate([alo[...] + ylo[...], ahi[...] + yhi[...]], axis=1)
  h = dv * t + b_ref[...]
  hh = jnp.concatenate([h, jnp.ones((_BN, 1), jnp.float32)], axis=1)
  gids = lax.broadcasted_iota(jnp.int32, (G, _BN), 0)
  oht = jnp.where(batch[...][0] == gids, 1.0, 0.0)
  acc[...] += lax.dot_general(oht, hh, (((1,), (0,)), ((), ())),
                              preferred_element_type=jnp.float32)

  @pl.when(i == _NB - 1)
  def _():
    sums = acc[...][:, :H]
    cnts = jnp.maximum(acc[...][:, H:], 1.0)
    out_ref[...] = (jnp.dot(sums / cnts, wl[...],
                            preferred_element_type=jnp.float32) + bl[...])


def _final(alo, ahi, ylo, yhi, dinv, b3, batch, wl, bl):
  spec = pl.BlockSpec((_BN, HH), lambda i: (i, 0))
  return pl.pallas_call(
      _final_body,
      grid=(_NB,),
      in_specs=[
          spec, spec, spec, spec,
          pl.BlockSpec((1, 1, _BN), lambda i: (i, 0, 0)),
          pl.BlockSpec((1, H), lambda i: (0, 0)),
          pl.BlockSpec((1, 1, _BN), lambda i: (i, 0, 0)),
          pl.BlockSpec((H, 3), lambda i: (0, 0)),
          pl.BlockSpec((1, 3), lambda i: (0, 0)),
      ],
      out_specs=pl.BlockSpec((G, 3), lambda i: (0, 0)),
      out_shape=jax.ShapeDtypeStruct((G, 3), jnp.float32),
      scratch_shapes=[pltpu.VMEM((G, H + 1), jnp.float32)],
  )(alo, ahi, ylo, yhi, dinv, b3, batch, wl, bl)


@jax.jit
def kernel(x, edge_index, batch, W1, b1, W2, b2, W3, b3, Wl, bl):
  ei3 = edge_index.reshape(2, NROWS, EPR)
  d0, d1 = _deg_kernel(ei3)
  y1lo, y1hi, dinv = _mm1(x, W1, d0.reshape(_NB, 1, _BN), d1.reshape(_NB, 1, _BN))
  a1lo, a1hi = _agg_kernel(ei3, y1lo, y1hi)
  y2lo, y2hi = _combine(a1lo, a1hi, y1lo, y1hi, dinv, b1.reshape(1, H), W2)
  a2lo, a2hi = _agg_kernel(ei3, y2lo, y2hi)
  y3lo, y3hi = _combine(a2lo, a2hi, y2lo, y2hi, dinv, b2.reshape(1, H), W3)
  a3lo, a3hi = _agg_kernel(ei3, y3lo, y3hi)
  return _final(a3lo, a3hi, y3lo, y3hi, dinv, b3.reshape(1, H),
                batch.reshape(_NB, 1, _BN), Wl, bl.reshape(1, 3))


# speedup vs baseline: 1.0636x; 1.0636x over previous
import functools

import jax
import jax.numpy as jnp
from jax import lax
from jax.experimental import pallas as pl
from jax.experimental.pallas import tpu as pltpu
from jax.experimental.pallas import tpu_sc as plsc

N = 50000
E = 800000
F_IN = 840
H = 64
HH = 32
G = 128
EPR = 128
NROWS = E // EPR
NS = 16
NC = 2

_mesh = plsc.VectorSubcoreMesh(core_axis_name="c", subcore_axis_name="s")
_sc_params = pltpu.CompilerParams(use_tc_tiling_on_sc=False)


def _fill(ref, n, val):
  v = jnp.full((16,), val, ref.dtype)
  def body(i, _):
    ref[pl.ds(i * 16, 16)] = v
    return 0
  lax.fori_loop(0, n // 16, body, 0)


def _fill2d(ref, rows, val):
  v = jnp.full((16,), val, ref.dtype)
  def body(i, _):
    ref[i, pl.ds(0, 16)] = v
    ref[i, pl.ds(16, 16)] = v
    return 0
  lax.fori_loop(0, rows, body, 0)


def _deg_body(ei3, deg0, deg1, dstb, ones_v, zb, deg_sh,
              si0, si1, si2, si3, ss0, ss1, ss2, ss3, ss4, ss5, ss6, ss7):
  c = lax.axis_index("c")
  s = lax.axis_index("s")
  w = s * NC + c
  si = (si0, si1, si2, si3)
  ss = (ss0, ss1, ss2, ss3, ss4, ss5, ss6, ss7)

  _fill(ones_v, EPR, 1.0)
  _fill(zb, EPR, 0.0)

  for k in range(25):
    @pl.when((s < 15) | (k < 15))
    def _():
      pltpu.sync_copy(zb, deg_sh.at[pl.ds(s * 3200 + k * 128, 128)])
  @pl.when(s == 15)
  def _():
    pltpu.sync_copy(zb.at[pl.ds(0, 80)], deg_sh.at[pl.ds(49920, 80)])
  plsc.subcore_barrier()

  nr = jnp.where(w < 10, 196, 195)
  start = jnp.where(w < 10, 196 * w, 195 * w + 10)
  row_of = lambda j: start + jnp.minimum(j, nr - 1)

  def issue_idx(j, slot, sem):
    pltpu.async_copy(ei3.at[1, pl.ds(row_of(j), 1), :],
                     dstb.at[pl.ds(slot, 1), :], sem)

  def wait_idx(slot, sem):
    pltpu.make_async_copy(ei3.at[1, pl.ds(0, 1), :],
                          dstb.at[pl.ds(slot, 1), :], sem).wait()

  def wait_sc(slot, sem):
    pltpu.make_async_copy(ones_v, deg_sh.at[dstb.at[slot]], sem).wait()

  for b in range(4):
    issue_idx(jnp.int32(b), b, si[b])

  NRS = 200

  def group8(g, _):
    for b in range(8):
      j = g * 8 + b
      wait_idx(b, si[b % 4])
      @pl.when(j < nr)
      def _():
        pltpu.async_copy(ones_v, deg_sh.at[dstb.at[b]], ss[b], add=True)
      @pl.when((j >= 4) & (j - 4 < nr))
      def _():
        wait_sc((b + 4) % 8, ss[(b + 4) % 8])
      issue_idx(j + 4, (b + 4) % 8, si[b % 4])
    return 0
  lax.fori_loop(0, NRS // 8, group8, 0)

  for b in range(4):
    wait_idx(b, si[b])

  plsc.subcore_barrier()

  def copy_out(dst_ref):
    def cbody(k, _):
      o = s * 3200 + k * 128
      pltpu.sync_copy(deg_sh.at[pl.ds(o, 128)], zb)
      pltpu.sync_copy(zb, dst_ref.at[pl.ds(o, 128)])
      return 0
    nchunk = jnp.where(s < 15, 25, 15)
    lax.fori_loop(0, nchunk, cbody, 0)
    @pl.when(s == 15)
    def _():
      pltpu.sync_copy(deg_sh.at[pl.ds(49920, 80)], zb.at[pl.ds(0, 80)])
      pltpu.sync_copy(zb.at[pl.ds(0, 80)], dst_ref.at[pl.ds(49920, 80)])

  @pl.when(c == 0)
  def _():
    copy_out(deg0)
  @pl.when(c == 1)
  def _():
    copy_out(deg1)


_deg_kernel = functools.partial(
    pl.kernel, _deg_body,
    out_type=(jax.ShapeDtypeStruct((N,), jnp.float32),
              jax.ShapeDtypeStruct((N,), jnp.float32)),
    mesh=_mesh,
    scratch_types=[
        pltpu.VMEM((8, EPR), jnp.int32),
        pltpu.VMEM((EPR,), jnp.float32),
        pltpu.VMEM((EPR,), jnp.float32),
        pltpu.VMEM_SHARED((N,), jnp.float32),
    ] + [pltpu.SemaphoreType.DMA] * 12,
    compiler_params=_sc_params,
)()


NRE = 392


def _agg_body(ei3, ylo, yhi, outlo, outhi, srcb, dstb, rows, zbuf, agg_sh,
              sg0, sg1, sg2, sg3, ss0, ss1, ss2, ss3, si0, si1, si2, si3):
  c = lax.axis_index("c")
  s = lax.axis_index("s")
  sg = (sg0, sg1, sg2, sg3)
  ss = (ss0, ss1, ss2, ss3)
  si = (si0, si1, si2, si3)

  _fill2d(zbuf, EPR, 0.0)

  r0 = s * 3128
  def zbody(k, _):
    pltpu.sync_copy(zbuf, agg_sh.at[pl.ds(r0 + k * 128, 128), :])
    return 0
  lax.fori_loop(0, 24, zbody, 0)
  @pl.when(s < 15)
  def _():
    pltpu.sync_copy(zbuf.at[pl.ds(0, 56), :],
                    agg_sh.at[pl.ds(r0 + 3072, 56), :])
  @pl.when(s == 15)
  def _():
    pltpu.sync_copy(zbuf.at[pl.ds(0, 8), :],
                    agg_sh.at[pl.ds(r0 + 3072, 8), :])
  plsc.subcore_barrier()

  nr = jnp.where(s < 10, 391, 390)
  start = jnp.where(s < 10, 391 * s, 390 * s + 10)
  row_of = lambda j: start + jnp.minimum(j, nr - 1)

  def do_edges(y_ref):
    def issue_idx(j, slot, sem):
      pltpu.async_copy(ei3.at[0, pl.ds(row_of(j), 1), :],
                       srcb.at[pl.ds(slot, 1), :], sem)
      pltpu.async_copy(ei3.at[1, pl.ds(row_of(j), 1), :],
                       dstb.at[pl.ds(slot, 1), :], sem)

    def wait_idx(slot, sem):
      pltpu.make_async_copy(ei3.at[0, pl.ds(0, 1), :],
                            srcb.at[pl.ds(slot, 1), :], sem).wait()
      pltpu.make_async_copy(ei3.at[1, pl.ds(0, 1), :],
                            dstb.at[pl.ds(slot, 1), :], sem).wait()

    def issue_gather(islot, rslot, sem):
      pltpu.async_copy(y_ref.at[srcb.at[islot]], rows.at[rslot], sem)

    def wait_gather(islot, rslot, sem):
      pltpu.make_async_copy(y_ref.at[srcb.at[islot]], rows.at[rslot],
                            sem).wait()

    def issue_sc(rslot, islot, sem):
      pltpu.async_copy(rows.at[rslot], agg_sh.at[dstb.at[islot]], sem,
                       add=True)

    def wait_sc(rslot, islot, sem):
      pltpu.make_async_copy(rows.at[rslot], agg_sh.at[dstb.at[islot]],
                            sem).wait()

    for b in range(3):
      issue_idx(jnp.int32(b), b, si[b])
    for b in range(3):
      wait_idx(b, si[b])
      issue_gather(b, b, sg[b])
    issue_idx(jnp.int32(3), 3, si[3])

    def group8(g, _):
      for b in range(8):
        j = g * 8 + b
        wait_gather(b, b % 4, sg[b % 4])
        @pl.when((j >= 1) & (j - 1 < nr))
        def _():
          wait_sc((b + 3) % 4, (b + 7) % 8, ss[(b + 3) % 4])
        wait_idx((b + 3) % 8, si[(b + 3) % 4])
        issue_gather((b + 3) % 8, (b + 3) % 4, sg[(b + 3) % 4])
        @pl.when(j < nr)
        def _():
          issue_sc(b % 4, b, ss[b % 4])
        issue_idx(j + 4, (b + 4) % 8, si[b % 4])
      return 0
    lax.fori_loop(0, NRE // 8, group8, 0)

    for b in range(3):
      wait_gather(b, b % 4, sg[b % 4])
    wait_idx(3, si[3])

  @pl.when(c == 0)
  def _():
    do_edges(ylo)
  @pl.when(c == 1)
  def _():
    do_edges(yhi)

  plsc.subcore_barrier()

  def copy_out(dst_ref):
    def cbody(k, _):
      o = r0 + k * 128
      pltpu.sync_copy(agg_sh.at[pl.ds(o, 128), :], zbuf)
      pltpu.sync_copy(zbuf, dst_ref.at[pl.ds(o, 128), :])
      return 0
    lax.fori_loop(0, 24, cbody, 0)
    @pl.when(s < 15)
    def _():
      pltpu.sync_copy(agg_sh.at[pl.ds(r0 + 3072, 56), :],
                      zbuf.at[pl.ds(0, 56), :])
      pltpu.sync_copy(zbuf.at[pl.ds(0, 56), :],
                      dst_ref.at[pl.ds(r0 + 3072, 56), :])
    @pl.when(s == 15)
    def _():
      pltpu.sync_copy(agg_sh.at[pl.ds(r0 + 3072, 8), :],
                      zbuf.at[pl.ds(0, 8), :])
      pltpu.sync_copy(zbuf.at[pl.ds(0, 8), :],
                      dst_ref.at[pl.ds(r0 + 3072, 8), :])

  @pl.when(c == 0)
  def _():
    copy_out(outlo)
  @pl.when(c == 1)
  def _():
    copy_out(outhi)


_agg_kernel = functools.partial(
    pl.kernel, _agg_body,
    out_type=(jax.ShapeDtypeStruct((N, HH), jnp.float32),
              jax.ShapeDtypeStruct((N, HH), jnp.float32)),
    mesh=_mesh,
    scratch_types=[
        pltpu.VMEM((8, EPR), jnp.int32),
        pltpu.VMEM((8, EPR), jnp.int32),
        pltpu.VMEM((4, EPR, HH), jnp.float32),
        pltpu.VMEM((EPR, HH), jnp.float32),
        pltpu.VMEM_SHARED((N, HH), jnp.float32),
    ] + [pltpu.SemaphoreType.DMA] * 12,
    compiler_params=_sc_params,
)()


_BN1 = 400


def _mm1_body(x_ref, w_ref, d0_ref, d1_ref, ylo_ref, yhi_ref, dinv_ref):
  deg = 1.0 + d0_ref[...] + d1_ref[...]
  dinv = lax.rsqrt(deg)
  xw = jnp.dot(x_ref[...], w_ref[...], preferred_element_type=jnp.float32)
  y = xw * dinv
  ylo_ref[...] = y[:, :HH]
  yhi_ref[...] = y[:, HH:]
  dinv_ref[...] = dinv


def _mm1(xv, w1, d0, d1):
  nb = N // _BN1
  return pl.pallas_call(
      _mm1_body,
      grid=(nb,),
      in_specs=[
          pl.BlockSpec((_BN1, F_IN), lambda i: (i, 0)),
          pl.BlockSpec((F_IN, H), lambda i: (0, 0)),
          pl.BlockSpec((_BN1, 1), lambda i: (i, 0)),
          pl.BlockSpec((_BN1, 1), lambda i: (i, 0)),
      ],
      out_specs=[
          pl.BlockSpec((_BN1, HH), lambda i: (i, 0)),
          pl.BlockSpec((_BN1, HH), lambda i: (i, 0)),
          pl.BlockSpec((_BN1, 1), lambda i: (i, 0)),
      ],
      out_shape=[
          jax.ShapeDtypeStruct((N, HH), jnp.float32),
          jax.ShapeDtypeStruct((N, HH), jnp.float32),
          jax.ShapeDtypeStruct((N, 1), jnp.float32),
      ],
  )(xv, w1, d0, d1)


_BN2 = 1000


def _comb_body(alo, ahi, ylo, yhi, dinv, b_ref, w_ref, olo, ohi):
  dv = dinv[...]
  t = jnp.concatenate([alo[...] + ylo[...], ahi[...] + yhi[...]], axis=1)
  h = jnp.maximum(dv * t + b_ref[...], 0.0)
  yn = jnp.dot(h, w_ref[...], preferred_element_type=jnp.float32) * dv
  olo[...] = yn[:, :HH]
  ohi[...] = yn[:, HH:]


def _combine(alo, ahi, ylo, yhi, dinv, b, w):
  nb = N // _BN2
  spec = pl.BlockSpec((_BN2, HH), lambda i: (i, 0))
  return pl.pallas_call(
      _comb_body,
      grid=(nb,),
      in_specs=[
          spec, spec, spec, spec,
          pl.BlockSpec((_BN2, 1), lambda i: (i, 0)),
          pl.BlockSpec((1, H), lambda i: (0, 0)),
          pl.BlockSpec((H, H), lambda i: (0, 0)),
      ],
      out_specs=[spec, spec],
      out_shape=[
          jax.ShapeDtypeStruct((N, HH), jnp.float32),
          jax.ShapeDtypeStruct((N, HH), jnp.float32),
      ],
  )(alo, ahi, ylo, yhi, dinv, b, w)


_BN3 = 1000
_NB3 = N // _BN3


def _final_body(alo, ahi, ylo, yhi, dinv, b_ref, batch, wl, bl, out_ref, acc):
  i = pl.program_id(0)

  @pl.when(i == 0)
  def _():
    acc[...] = jnp.zeros((G, H + 1), jnp.float32)

  t = jnp.concatenate([alo[...] + ylo[...], ahi[...] + yhi[...]], axis=1)
  h = dinv[...] * t + b_ref[...]
  hh = jnp.concatenate([h, jnp.ones((_BN3, 1), jnp.float32)], axis=1)
  gids = lax.broadcasted_iota(jnp.int32, (_BN3, G), 1)
  oh = jnp.where(batch[...] == gids, 1.0, 0.0)
  acc[...] += lax.dot_general(oh, hh, (((0,), (0,)), ((), ())),
                              preferred_element_type=jnp.float32)

  @pl.when(i == _NB3 - 1)
  def _():
    sums = acc[...][:, :H]
    cnts = jnp.maximum(acc[...][:, H:], 1.0)
    out_ref[...] = (jnp.dot(sums / cnts, wl[...],
                            preferred_element_type=jnp.float32) + bl[...])


def _final(alo, ahi, ylo, yhi, dinv, b3, batch, wl, bl):
  spec = pl.BlockSpec((_BN3, HH), lambda i: (i, 0))
  return pl.pallas_call(
      _final_body,
      grid=(_NB3,),
      in_specs=[
          spec, spec, spec, spec,
          pl.BlockSpec((_BN3, 1), lambda i: (i, 0)),
          pl.BlockSpec((1, H), lambda i: (0, 0)),
          pl.BlockSpec((_BN3, 1), lambda i: (i, 0)),
          pl.BlockSpec((H, 3), lambda i: (0, 0)),
          pl.BlockSpec((1, 3), lambda i: (0, 0)),
      ],
      out_specs=pl.BlockSpec((G, 3), lambda i: (0, 0)),
      out_shape=jax.ShapeDtypeStruct((G, 3), jnp.float32),
      scratch_shapes=[pltpu.VMEM((G, H + 1), jnp.float32)],
  )(alo, ahi, ylo, yhi, dinv, b3, batch, wl, bl)


@jax.jit
def kernel(x, edge_index, batch, W1, b1, W2, b2, W3, b3, Wl, bl):
  ei3 = edge_index.reshape(2, NROWS, EPR)
  d0, d1 = _deg_kernel(ei3)
  y1lo, y1hi, dinv = _mm1(x, W1, d0.reshape(N, 1), d1.reshape(N, 1))
  a1lo, a1hi = _agg_kernel(ei3, y1lo, y1hi)
  y2lo, y2hi = _combine(a1lo, a1hi, y1lo, y1hi, dinv, b1.reshape(1, H), W2)
  a2lo, a2hi = _agg_kernel(ei3, y2lo, y2hi)
  y3lo, y3hi = _combine(a2lo, a2hi, y2lo, y2hi, dinv, b2.reshape(1, H), W3)
  a3lo, a3hi = _agg_kernel(ei3, y3lo, y3hi)
  return _final(a3lo, a3hi, y3lo, y3hi, dinv, b3.reshape(1, H),
                batch.reshape(N, 1), Wl, bl.reshape(1, 3))
